# baseline (speedup 1.0000x reference)
import jax
import jax.numpy as jnp
from jax import lax
from jax.experimental import pallas as pl
from jax.experimental.pallas import tpu as pltpu
from jax.experimental.pallas import tpu_sc as plsc

N = 10000
E = 320000
D = 128
TEMP = 0.1
EPS = 1e-20

NP = 10240
EP = 321536
ROWS = 157
EPT = ROWS * 128
NPT = NP // 16
DH = D // 2
UROWS = EP // 128

_NEG = -1e30

_SC_PARAMS = pltpu.CompilerParams(needs_layout_passes=False,
                                  use_tc_tiling_on_sc=False)
_SC_MESH = plsc.VectorSubcoreMesh(core_axis_name="c", subcore_axis_name="s")


def _tc_body(x_ref, w_ref, u_ref, s_ref, g_ref):
    s_ref[...] = jnp.dot(x_ref[...], w_ref[...],
                         preferred_element_type=jnp.float32)
    u = u_ref[...]
    g_ref[...] = -jnp.log(-jnp.log(u + EPS) + EPS)


_tc_stage = pl.pallas_call(
    _tc_body,
    out_shape=(
        jax.ShapeDtypeStruct((N, 8), jnp.float32),
        jax.ShapeDtypeStruct((UROWS, 128), jnp.float32),
    ),
)


def _sca_body(srcR, dstR, gR, s1_hbm, s2_hbm, alphaR, pub_hbm,
              src2, dst2, aex, m_t, s_t, buf1, buf2,
              shift_k, shift_v, tm, ts, macc, sacc,
              merged_sp):
    cid = lax.axis_index("c")
    tid = lax.axis_index("s")
    inv_temp = jnp.float32(1.0 / TEMP)

    pltpu.sync_copy(srcR.at[tid], src2)
    pltpu.sync_copy(dstR.at[tid], dst2)
    pltpu.sync_copy(gR.at[tid], aex)
    pltpu.sync_copy(s1_hbm, buf1)
    pltpu.sync_copy(s2_hbm, buf2)

    def _init(i, c):
        m_t[pl.ds(i * 16, 16)] = jnp.full((16,), _NEG, jnp.float32)
        s_t[pl.ds(i * 16, 16)] = jnp.zeros((16,), jnp.float32)
        return c
    lax.fori_loop(0, NP // 16, _init, 0)

    shift_k[pl.ds(0, 16)] = jnp.full((16,), -1, jnp.int32)
    shift_v[pl.ds(0, 16)] = jnp.full((16,), _NEG, jnp.float32)

    def _p1(r, c):
        for q in range(8):
            sl = pl.ds(q * 16, 16)
            src16 = src2[r, sl]
            dst16 = dst2[r, sl]
            g16 = aex[r, sl]
            a16 = (plsc.load_gather(buf1, [src16])
                   + plsc.load_gather(buf2, [dst16]) + g16) * inv_temp
            aex[r, sl] = a16
            ks, vs = plsc.sort_key_val(src16, a16)
            shift_k[pl.ds(16, 16)] = ks
            shift_v[pl.ds(16, 16)] = vs
            for s in (1, 2, 4, 8):
                kk = shift_k[pl.ds(16 - s, 16)]
                vv = shift_v[pl.ds(16 - s, 16)]
                vs = jnp.where(kk == ks, jnp.maximum(vs, vv), vs)
                shift_v[pl.ds(16, 16)] = vs
            _u, last = plsc.scan_count(ks)
            cur = plsc.load_gather(m_t, [ks], mask=last)
            plsc.store_scatter(m_t, [ks], jnp.maximum(cur, vs), mask=last)
        return c
    lax.fori_loop(0, ROWS, _p1, 0)

    shift_v[pl.ds(0, 16)] = jnp.zeros((16,), jnp.float32)

    def _p2(r, c):
        for q in range(8):
            sl = pl.ds(q * 16, 16)
            src16 = src2[r, sl]
            ex = jnp.exp(aex[r, sl] - plsc.load_gather(m_t, [src16]))
            aex[r, sl] = ex
            ks, vs = plsc.sort_key_val(src16, ex)
            shift_k[pl.ds(16, 16)] = ks
            shift_v[pl.ds(16, 16)] = vs
            for s in (1, 2, 4, 8):
                kk = shift_k[pl.ds(16 - s, 16)]
                vv = shift_v[pl.ds(16 - s, 16)]
                vs = vs + jnp.where(kk == ks, vv, jnp.float32(0.0))
                shift_v[pl.ds(16, 16)] = vs
            _u, last = plsc.scan_count(ks)
            plsc.addupdate_scatter(s_t, [ks], vs, mask=last)
        return c
    lax.fori_loop(0, ROWS, _p2, 0)

    pltpu.sync_copy(m_t, pub_hbm.at[cid, tid, 0])
    pltpu.sync_copy(s_t, pub_hbm.at[cid, tid, 1])
    plsc.subcore_barrier()

    base = tid * NPT

    def _macc_init(i, c):
        macc[pl.ds(i * 16, 16)] = jnp.full((16,), _NEG, jnp.float32)
        sacc[pl.ds(i * 16, 16)] = jnp.zeros((16,), jnp.float32)
        return c
    lax.fori_loop(0, NPT // 16, _macc_init, 0)

    for t in range(16):
        pltpu.sync_copy(pub_hbm.at[cid, t, 0, pl.ds(base, NPT)], tm)
        pltpu.sync_copy(pub_hbm.at[cid, t, 1, pl.ds(base, NPT)], ts)

        def _merge(i, c):
            sl = pl.ds(i * 16, 16)
            m_old = macc[sl]
            m_new = jnp.maximum(m_old, tm[sl])
            sacc[sl] = (sacc[sl] * jnp.exp(m_old - m_new)
                        + ts[sl] * jnp.exp(tm[sl] - m_new))
            macc[sl] = m_new
            return c
        lax.fori_loop(0, NPT // 16, _merge, 0)

    def _invs(i, c):
        sl = pl.ds(i * 16, 16)
        sacc[sl] = jnp.float32(1.0) / (sacc[sl] + jnp.float32(1e-16))
        return c
    lax.fori_loop(0, NPT // 16, _invs, 0)

    pltpu.sync_copy(macc, merged_sp.at[0, pl.ds(base, NPT)])
    pltpu.sync_copy(sacc, merged_sp.at[1, pl.ds(base, NPT)])
    plsc.subcore_barrier()

    pltpu.sync_copy(merged_sp.at[0], buf1)
    pltpu.sync_copy(merged_sp.at[1], buf2)

    def _ft(i, c):
        sl = pl.ds(i * 16, 16)
        m_t[sl] = jnp.exp(m_t[sl] - buf1[sl]) * buf2[sl]
        return c
    lax.fori_loop(0, NP // 16, _ft, 0)

    def _al(r, c):
        for q in range(8):
            sl = pl.ds(q * 16, 16)
            aex[r, sl] = aex[r, sl] * plsc.load_gather(m_t, [src2[r, sl]])
        return c
    lax.fori_loop(0, ROWS, _al, 0)

    @pl.when(cid == 0)
    def _():
        pltpu.sync_copy(aex, alphaR.at[tid])


_sca_stage = pl.kernel(
    _sca_body,
    out_type=(jax.ShapeDtypeStruct((16, ROWS, 128), jnp.float32),
              jax.ShapeDtypeStruct((2, 16, 2, NP), jnp.float32)),
    mesh=_SC_MESH,
    compiler_params=_SC_PARAMS,
    scratch_types=[
        pltpu.VMEM((ROWS, 128), jnp.int32),
        pltpu.VMEM((ROWS, 128), jnp.int32),
        pltpu.VMEM((ROWS, 128), jnp.float32),
        pltpu.VMEM((NP,), jnp.float32),
        pltpu.VMEM((NP,), jnp.float32),
        pltpu.VMEM((NP,), jnp.float32),
        pltpu.VMEM((NP,), jnp.float32),
        pltpu.VMEM((32,), jnp.int32),
        pltpu.VMEM((32,), jnp.float32),
        pltpu.VMEM((NPT,), jnp.float32),
        pltpu.VMEM((NPT,), jnp.float32),
        pltpu.VMEM((NPT,), jnp.float32),
        pltpu.VMEM((NPT,), jnp.float32),
        pltpu.VMEM_SHARED((2, NP), jnp.float32),
    ],
)


def _scb_body(x2_hbm, srcR, dstR, alphaR, outp_hbm,
              gidx2, dst2, aex, rows, gsem, ssem, out_sp):
    cid = lax.axis_index("c")
    tid = lax.axis_index("s")

    pltpu.sync_copy(srcR.at[tid], gidx2)
    pltpu.sync_copy(dstR.at[tid], dst2)
    pltpu.sync_copy(alphaR.at[tid], aex)

    def _zrows(i, c):
        for q in range(4):
            rows[0, i, pl.ds(q * 16, 16)] = jnp.zeros((16,), jnp.float32)
        return c
    lax.fori_loop(0, 128, _zrows, 0)
    for b in range(NPT // 128):
        pltpu.sync_copy(rows.at[0],
                        out_sp.at[pl.ds(tid * NPT + b * 128, 128)])

    def _gx(r, c):
        for q in range(8):
            sl = pl.ds(q * 16, 16)
            gidx2[r, sl] = jnp.minimum(gidx2[r, sl] * 2 + cid,
                                       jnp.int32(2 * N - 1))
        return c
    lax.fori_loop(0, ROWS, _gx, 0)
    plsc.subcore_barrier()

    pltpu.async_copy(x2_hbm.at[gidx2.at[0]], rows.at[0], gsem)
    pltpu.async_copy(x2_hbm.at[gidx2.at[1]], rows.at[1], gsem)

    def _p4(r, c):
        b = r % 3
        pltpu.make_async_copy(x2_hbm.at[gidx2.at[r]], rows.at[b], gsem).wait()

        @pl.when(r + 2 < ROWS)
        def _pref():
            b2 = (r + 2) % 3

            @pl.when(r >= 1)
            def _free():
                pltpu.make_async_copy(rows.at[b2],
                                      out_sp.at[dst2.at[r - 1]], ssem).wait()
            pltpu.async_copy(x2_hbm.at[gidx2.at[r + 2]], rows.at[b2], gsem)

        r16 = jnp.full((16,), r, jnp.int32)

        @plsc.parallel_loop(0, 128, unroll=8)
        def _scale(e):
            av = plsc.load_gather(aex, [r16, jnp.full((16,), e, jnp.int32)])
            for dq in range(4):
                dsl = pl.ds(dq * 16, 16)
                rows[b, e, dsl] = rows[b, e, dsl] * av

        pltpu.async_copy(rows.at[b], out_sp.at[dst2.at[r]], ssem, add=True)
        return c
    lax.fori_loop(0, ROWS, _p4, 0)
    for rr in (ROWS - 3, ROWS - 2, ROWS - 1):
        pltpu.make_async_copy(rows.at[rr % 3],
                              out_sp.at[dst2.at[rr]], ssem).wait()
    plsc.subcore_barrier()

    pltpu.sync_copy(out_sp.at[pl.ds(tid * NPT, NPT)],
                    outp_hbm.at[pl.ds(tid * NPT, NPT), cid])


_scb_stage = pl.kernel(
    _scb_body,
    out_type=jax.ShapeDtypeStruct((NP, 2, DH), jnp.float32),
    mesh=_SC_MESH,
    compiler_params=_SC_PARAMS,
    scratch_types=[
        pltpu.VMEM((ROWS, 128), jnp.int32),
        pltpu.VMEM((ROWS, 128), jnp.int32),
        pltpu.VMEM((ROWS, 128), jnp.float32),
        pltpu.VMEM((3, 128, DH), jnp.float32),
        pltpu.SemaphoreType.DMA,
        pltpu.SemaphoreType.DMA,
        pltpu.VMEM_SHARED((NP, DH), jnp.float32),
    ],
)


@jax.jit
def kernel(x, edge_index, batch, att_weight):
    src = edge_index[0]
    dst = edge_index[1]
    x2 = x.reshape(N * 2, DH)

    pad_ids = (N + (jnp.arange(EP - E, dtype=jnp.int32) % 8)).astype(jnp.int32)
    srcR = jnp.concatenate([src, pad_ids]).reshape(16, ROWS, 128)
    dstR = jnp.concatenate([dst, pad_ids]).reshape(16, ROWS, 128)

    u = jax.random.uniform(jax.random.key(42), (E,), dtype=jnp.float32)
    up = jnp.concatenate([u, jnp.full((EP - E,), 0.5, jnp.float32)])
    up = up.reshape(UROWS, 128)

    w1 = att_weight[0, :D]
    w2 = att_weight[0, D:]
    Wp = jnp.zeros((D, 8), jnp.float32).at[:, 0].set(w1).at[:, 1].set(w2)

    s_pad, g2 = _tc_stage(x, Wp, up)
    gR = g2.reshape(16, ROWS, 128)

    s1p = jnp.pad(s_pad[:, 0], (0, NP - N))
    s2p = jnp.pad(s_pad[:, 1], (0, NP - N))
    alphaR, _pub = _sca_stage(srcR, dstR, gR, s1p, s2p)
    outp = _scb_stage(x2, srcR, dstR, alphaR)
    out = outp.reshape(NP, D)[:N]

    score = jnp.zeros((N,), out.dtype)
    perm = jnp.arange(N, dtype=jnp.int32)
    return (out, edge_index, batch, perm, score)


# speedup vs baseline: 1.0470x; 1.0470x over previous
import jax
import jax.numpy as jnp
from jax import lax
from jax.experimental import pallas as pl
from jax.experimental.pallas import tpu as pltpu
from jax.experimental.pallas import tpu_sc as plsc

N = 10000
E = 320000
D = 128
TEMP = 0.1
EPS = 1e-20

NP = 10240
EP = 321536
ROWS = 157
EPT = ROWS * 128
NPT = NP // 16
DH = D // 2
UROWS = EP // 128

_NEG = -1e30

_SC_PARAMS = pltpu.CompilerParams(needs_layout_passes=False,
                                  use_tc_tiling_on_sc=False)
_SC_MESH = plsc.VectorSubcoreMesh(core_axis_name="c", subcore_axis_name="s")


def _tc_body(x_ref, w_ref, u_ref, s_ref, g_ref):
    s_ref[...] = jnp.dot(x_ref[...], w_ref[...],
                         preferred_element_type=jnp.float32)
    u = u_ref[...]
    g_ref[...] = -jnp.log(-jnp.log(u + EPS) + EPS)


_tc_stage = pl.pallas_call(
    _tc_body,
    out_shape=(
        jax.ShapeDtypeStruct((N, 8), jnp.float32),
        jax.ShapeDtypeStruct((UROWS, 128), jnp.float32),
    ),
)


def _sca_body(srcR, dstR, gR, s1_hbm, s2_hbm, alphaR, pub_hbm,
              src2, dst2, aex, m_t, s_t, buf1, buf2,
              shift_k, shift_v, tm, ts, macc, sacc,
              merged_sp):
    cid = lax.axis_index("c")
    tid = lax.axis_index("s")
    inv_temp = jnp.float32(1.0 / TEMP)

    pltpu.sync_copy(srcR.at[tid], src2)
    pltpu.sync_copy(dstR.at[tid], dst2)
    pltpu.sync_copy(gR.at[tid], aex)
    pltpu.sync_copy(s1_hbm, buf1)
    pltpu.sync_copy(s2_hbm, buf2)

    def _init(i, c):
        m_t[pl.ds(i * 16, 16)] = jnp.full((16,), _NEG, jnp.float32)
        s_t[pl.ds(i * 16, 16)] = jnp.zeros((16,), jnp.float32)
        return c
    lax.fori_loop(0, NP // 16, _init, 0)

    shift_k[pl.ds(0, 16)] = jnp.full((16,), -1, jnp.int32)
    shift_v[pl.ds(0, 16)] = jnp.full((16,), _NEG, jnp.float32)

    def _p1(r, c):
        for q in range(8):
            sl = pl.ds(q * 16, 16)
            src16 = src2[r, sl]
            dst16 = dst2[r, sl]
            g16 = aex[r, sl]
            a16 = (plsc.load_gather(buf1, [src16])
                   + plsc.load_gather(buf2, [dst16]) + g16) * inv_temp
            aex[r, sl] = a16
            ks, vs = plsc.sort_key_val(src16, a16)
            shift_k[pl.ds(16, 16)] = ks
            shift_v[pl.ds(16, 16)] = vs
            for s in (1, 2, 4, 8):
                kk = shift_k[pl.ds(16 - s, 16)]
                vv = shift_v[pl.ds(16 - s, 16)]
                vs = jnp.where(kk == ks, jnp.maximum(vs, vv), vs)
                shift_v[pl.ds(16, 16)] = vs
            _u, last = plsc.scan_count(ks)
            cur = plsc.load_gather(m_t, [ks], mask=last)
            plsc.store_scatter(m_t, [ks], jnp.maximum(cur, vs), mask=last)
        return c
    lax.fori_loop(0, ROWS, _p1, 0)

    shift_v[pl.ds(0, 16)] = jnp.zeros((16,), jnp.float32)

    def _p2(r, c):
        for q in range(8):
            sl = pl.ds(q * 16, 16)
            src16 = src2[r, sl]
            ex = jnp.exp(aex[r, sl] - plsc.load_gather(m_t, [src16]))
            aex[r, sl] = ex
            ks, vs = plsc.sort_key_val(src16, ex)
            shift_k[pl.ds(16, 16)] = ks
            shift_v[pl.ds(16, 16)] = vs
            for s in (1, 2, 4, 8):
                kk = shift_k[pl.ds(16 - s, 16)]
                vv = shift_v[pl.ds(16 - s, 16)]
                vs = vs + jnp.where(kk == ks, vv, jnp.float32(0.0))
                shift_v[pl.ds(16, 16)] = vs
            _u, last = plsc.scan_count(ks)
            plsc.addupdate_scatter(s_t, [ks], vs, mask=last)
        return c
    lax.fori_loop(0, ROWS, _p2, 0)

    pltpu.sync_copy(m_t, pub_hbm.at[cid, tid, 0])
    pltpu.sync_copy(s_t, pub_hbm.at[cid, tid, 1])
    plsc.subcore_barrier()

    base = tid * NPT

    def _macc_init(i, c):
        macc[pl.ds(i * 16, 16)] = jnp.full((16,), _NEG, jnp.float32)
        sacc[pl.ds(i * 16, 16)] = jnp.zeros((16,), jnp.float32)
        return c
    lax.fori_loop(0, NPT // 16, _macc_init, 0)

    for t in range(16):
        pltpu.sync_copy(pub_hbm.at[cid, t, 0, pl.ds(base, NPT)], tm)
        pltpu.sync_copy(pub_hbm.at[cid, t, 1, pl.ds(base, NPT)], ts)

        def _merge(i, c):
            sl = pl.ds(i * 16, 16)
            m_old = macc[sl]
            m_new = jnp.maximum(m_old, tm[sl])
            sacc[sl] = (sacc[sl] * jnp.exp(m_old - m_new)
                        + ts[sl] * jnp.exp(tm[sl] - m_new))
            macc[sl] = m_new
            return c
        lax.fori_loop(0, NPT // 16, _merge, 0)

    def _invs(i, c):
        sl = pl.ds(i * 16, 16)
        sacc[sl] = jnp.float32(1.0) / (sacc[sl] + jnp.float32(1e-16))
        return c
    lax.fori_loop(0, NPT // 16, _invs, 0)

    pltpu.sync_copy(macc, merged_sp.at[0, pl.ds(base, NPT)])
    pltpu.sync_copy(sacc, merged_sp.at[1, pl.ds(base, NPT)])
    plsc.subcore_barrier()

    pltpu.sync_copy(merged_sp.at[0], buf1)
    pltpu.sync_copy(merged_sp.at[1], buf2)

    def _ft(i, c):
        sl = pl.ds(i * 16, 16)
        m_t[sl] = jnp.exp(m_t[sl] - buf1[sl]) * buf2[sl]
        return c
    lax.fori_loop(0, NP // 16, _ft, 0)

    def _al(r, c):
        for q in range(8):
            sl = pl.ds(q * 16, 16)
            aex[r, sl] = aex[r, sl] * plsc.load_gather(m_t, [src2[r, sl]])
        return c
    lax.fori_loop(0, ROWS, _al, 0)

    @pl.when(cid == 0)
    def _():
        pltpu.sync_copy(aex, alphaR.at[tid])


_sca_stage = pl.kernel(
    _sca_body,
    out_type=(jax.ShapeDtypeStruct((16, ROWS, 128), jnp.float32),
              jax.ShapeDtypeStruct((2, 16, 2, NP), jnp.float32)),
    mesh=_SC_MESH,
    compiler_params=_SC_PARAMS,
    scratch_types=[
        pltpu.VMEM((ROWS, 128), jnp.int32),
        pltpu.VMEM((ROWS, 128), jnp.int32),
        pltpu.VMEM((ROWS, 128), jnp.float32),
        pltpu.VMEM((NP,), jnp.float32),
        pltpu.VMEM((NP,), jnp.float32),
        pltpu.VMEM((NP,), jnp.float32),
        pltpu.VMEM((NP,), jnp.float32),
        pltpu.VMEM((32,), jnp.int32),
        pltpu.VMEM((32,), jnp.float32),
        pltpu.VMEM((NPT,), jnp.float32),
        pltpu.VMEM((NPT,), jnp.float32),
        pltpu.VMEM((NPT,), jnp.float32),
        pltpu.VMEM((NPT,), jnp.float32),
        pltpu.VMEM_SHARED((2, NP), jnp.float32),
    ],
)


def _scb_body(x2_hbm, srcR, dstR, alphaR, outp_hbm,
              gidx2, dst2, aex, rows, gsem, ssem, out_sp):
    cid = lax.axis_index("c")
    tid = lax.axis_index("s")

    pltpu.sync_copy(srcR.at[tid], gidx2)
    pltpu.sync_copy(dstR.at[tid], dst2)
    pltpu.sync_copy(alphaR.at[tid], aex)

    def _zrows(i, c):
        for q in range(4):
            rows[0, i, pl.ds(q * 16, 16)] = jnp.zeros((16,), jnp.float32)
        return c
    lax.fori_loop(0, 128, _zrows, 0)
    for b in range(NPT // 128):
        pltpu.sync_copy(rows.at[0],
                        out_sp.at[pl.ds(tid * NPT + b * 128, 128)])

    def _gx(r, c):
        for q in range(8):
            sl = pl.ds(q * 16, 16)
            gidx2[r, sl] = jnp.minimum(gidx2[r, sl] * 2 + cid,
                                       jnp.int32(2 * N - 1))
        return c
    lax.fori_loop(0, ROWS, _gx, 0)
    plsc.subcore_barrier()

    pltpu.async_copy(x2_hbm.at[gidx2.at[0]], rows.at[0], gsem)
    pltpu.async_copy(x2_hbm.at[gidx2.at[1]], rows.at[1], gsem)

    def _p4(r, c):
        b = r % 3
        pltpu.make_async_copy(x2_hbm.at[gidx2.at[r]], rows.at[b], gsem).wait()

        @pl.when(r + 2 < ROWS)
        def _pref():
            b2 = (r + 2) % 3

            @pl.when(r >= 1)
            def _free():
                pltpu.make_async_copy(rows.at[b2],
                                      out_sp.at[dst2.at[r - 1]], ssem).wait()
            pltpu.async_copy(x2_hbm.at[gidx2.at[r + 2]], rows.at[b2], gsem)

        r16 = jnp.full((16,), r, jnp.int32)

        @plsc.parallel_loop(0, 128, unroll=8)
        def _scale(e):
            av = plsc.load_gather(aex, [r16, jnp.full((16,), e, jnp.int32)])
            for dq in range(4):
                dsl = pl.ds(dq * 16, 16)
                rows[b, e, dsl] = rows[b, e, dsl] * av

        pltpu.async_copy(rows.at[b], out_sp.at[dst2.at[r]], ssem, add=True)
        return c
    lax.fori_loop(0, ROWS, _p4, 0)
    for rr in (ROWS - 3, ROWS - 2, ROWS - 1):
        pltpu.make_async_copy(rows.at[rr % 3],
                              out_sp.at[dst2.at[rr]], ssem).wait()
    plsc.subcore_barrier()

    pltpu.sync_copy(out_sp.at[pl.ds(tid * NPT, NPT)],
                    outp_hbm.at[cid, pl.ds(tid * NPT, NPT)])


_scb_stage = pl.kernel(
    _scb_body,
    out_type=jax.ShapeDtypeStruct((2, NP, DH), jnp.float32),
    mesh=_SC_MESH,
    compiler_params=_SC_PARAMS,
    scratch_types=[
        pltpu.VMEM((ROWS, 128), jnp.int32),
        pltpu.VMEM((ROWS, 128), jnp.int32),
        pltpu.VMEM((ROWS, 128), jnp.float32),
        pltpu.VMEM((3, 128, DH), jnp.float32),
        pltpu.SemaphoreType.DMA,
        pltpu.SemaphoreType.DMA,
        pltpu.VMEM_SHARED((NP, DH), jnp.float32),
    ],
)


@jax.jit
def kernel(x, edge_index, batch, att_weight):
    src = edge_index[0]
    dst = edge_index[1]
    x2 = x.reshape(N * 2, DH)

    pad_ids = (N + (jnp.arange(EP - E, dtype=jnp.int32) % 8)).astype(jnp.int32)
    srcR = jnp.concatenate([src, pad_ids]).reshape(16, ROWS, 128)
    dstR = jnp.concatenate([dst, pad_ids]).reshape(16, ROWS, 128)

    u = jax.random.uniform(jax.random.key(42), (E,), dtype=jnp.float32)
    up = jnp.concatenate([u, jnp.full((EP - E,), 0.5, jnp.float32)])
    up = up.reshape(UROWS, 128)

    w1 = att_weight[0, :D]
    w2 = att_weight[0, D:]
    Wp = jnp.zeros((D, 8), jnp.float32).at[:, 0].set(w1).at[:, 1].set(w2)

    s_pad, g2 = _tc_stage(x, Wp, up)
    gR = g2.reshape(16, ROWS, 128)

    s1p = jnp.pad(s_pad[:, 0], (0, NP - N))
    s2p = jnp.pad(s_pad[:, 1], (0, NP - N))
    alphaR, _pub = _sca_stage(srcR, dstR, gR, s1p, s2p)
    outp = _scb_stage(x2, srcR, dstR, alphaR)
    out = outp.transpose(1, 0, 2).reshape(NP, D)[:N]

    score = jnp.zeros((N,), out.dtype)
    perm = jnp.arange(N, dtype=jnp.int32)
    return (out, edge_index, batch, perm, score)
